# baseline (speedup 1.0000x reference)
import jax
import jax.numpy as jnp
from jax import lax
from jax.experimental import pallas as pl
from jax.experimental.pallas import tpu as pltpu
from jax.experimental.pallas import tpu_sc as plsc

N = 10000
E = 160000
F_IN = 256
H = 512
OUT = 256
BLK = 128
NSUB = 16

EDGES_PER_SUB = E // NSUB
SEG_C = 80
SEG_CHUNKS = (E // NSUB) // SEG_C
SEG_PAIRS = (SEG_CHUNKS + 1) // 2
ROWS_MAIN = 624
ROWS_TAIL = N - ROWS_MAIN * NSUB

DEC_C = 40
DEC_W = 32
DEC_CHUNKS = (E // DEC_W) // DEC_C
DEC_PAIRS = (DEC_CHUNKS + 1) // 2

M_TILE = 1000
GRID_M = N // M_TILE


def _sc_mesh():
    return plsc.VectorSubcoreMesh(core_axis_name="c", subcore_axis_name="s")


def _segsum(x_blocks, src3, dst3):
    nblk = len(x_blocks)
    per_core = nblk // 2

    def body(*refs):
        x_refs = refs[:nblk]
        src_ref = refs[nblk]
        dst_ref = refs[nblk + 1]
        outs = refs[nblk + 2:2 * nblk + 2]
        acc, sidx, didx, rows0, rows1, sem0, sem1 = refs[2 * nblk + 2:]
        rows = (rows0, rows1)
        sems = (sem0, sem1)
        cid = lax.axis_index("c")
        sid = lax.axis_index("s")
        zvec = jnp.zeros((16,), jnp.float32)

        pltpu.sync_copy(src_ref.at[pl.ds(sid * EDGES_PER_SUB, EDGES_PER_SUB)],
                        sidx)
        pltpu.sync_copy(dst_ref.at[sid], didx)

        def one_block(xb, ob):
            def zrow(r, carry):
                for c8 in range(BLK // 16):
                    rows0[r, pl.ds(c8 * 16, 16)] = zvec
                return carry
            lax.fori_loop(0, SEG_C, zrow, 0)
            r0 = sid * ROWS_MAIN
            off = 0
            while off < ROWS_MAIN:
                span = min(SEG_C, ROWS_MAIN - off)
                pltpu.sync_copy(rows0.at[pl.ds(0, span)],
                                acc.at[pl.ds(r0 + off, span)])
                off += span
            pl.when(sid == 0)(lambda: pltpu.sync_copy(
                rows0.at[pl.ds(0, ROWS_TAIL)],
                acc.at[pl.ds(ROWS_MAIN * NSUB, ROWS_TAIL)]))
            plsc.subcore_barrier()

            for b in range(2):
                pltpu.async_copy(xb.at[sidx.at[pl.ds(b * SEG_C, SEG_C)]],
                                 rows[b], sems[b])

            def pair(j, carry):
                for b in range(2):
                    c = 2 * j + b

                    def slot(b=b, c=c):
                        pltpu.make_async_copy(
                            xb.at[sidx.at[pl.ds(0, SEG_C)]],
                            rows[b], sems[b]).wait()
                        pltpu.sync_copy(rows[b], acc.at[didx.at[c]],
                                        add=True)
                        nc = c + 2

                        def seg_prefetch():
                            pltpu.async_copy(
                                xb.at[sidx.at[pl.ds(nc * SEG_C, SEG_C)]],
                                rows[b], sems[b])
                        pl.when(nc < SEG_CHUNKS)(seg_prefetch)
                    pl.when(c < SEG_CHUNKS)(slot)
                return carry
            lax.fori_loop(0, SEG_PAIRS, pair, 0)
            plsc.subcore_barrier()
            pltpu.sync_copy(acc.at[pl.ds(r0, ROWS_MAIN)],
                            ob.at[pl.ds(r0, ROWS_MAIN)])
            pl.when(sid == 0)(lambda: pltpu.sync_copy(
                acc.at[pl.ds(ROWS_MAIN * NSUB, ROWS_TAIL)],
                ob.at[pl.ds(ROWS_MAIN * NSUB, ROWS_TAIL)]))

        for core in range(2):
            def run_core(core=core):
                for b in range(per_core):
                    blk = core * per_core + b
                    one_block(x_refs[blk], outs[blk])
            pl.when(cid == core)(run_core)

    f = pl.kernel(
        body,
        out_type=tuple(jax.ShapeDtypeStruct((N, BLK), jnp.float32)
                       for _ in range(nblk)),
        mesh=_sc_mesh(),
        scratch_types=[
            pltpu.VMEM_SHARED((N, BLK), jnp.float32),
            pltpu.VMEM((EDGES_PER_SUB,), jnp.int32),
            pltpu.VMEM((SEG_CHUNKS, SEG_C), jnp.int32),
            pltpu.VMEM((SEG_C, BLK), jnp.float32),
            pltpu.VMEM((SEG_C, BLK), jnp.float32),
            pltpu.SemaphoreType.DMA,
            pltpu.SemaphoreType.DMA,
        ],
    )
    return f(*x_blocks, src3, dst3)


def _decode(A, Bm, ai3, bi3):
    span = E // DEC_W

    def body(a_ref, b_ref, ai_ref, bi_ref, out_ref,
             aidx, bidx, ra0, ra1, rb0, rb1, g0, g1, w0, w1):
        cid = lax.axis_index("c")
        sid = lax.axis_index("s")
        wid = sid * 2 + cid
        ra = (ra0, ra1)
        rb = (rb0, rb1)
        gs = (g0, g1)
        ws = (w0, w1)
        base0 = wid * span

        pltpu.sync_copy(ai_ref.at[pl.ds(base0, span)], aidx)
        pltpu.sync_copy(bi_ref.at[pl.ds(base0, span)], bidx)
        for b in range(2):
            pltpu.async_copy(a_ref.at[aidx.at[pl.ds(b * DEC_C, DEC_C)]],
                             ra[b], gs[b])
            pltpu.async_copy(b_ref.at[bidx.at[pl.ds(b * DEC_C, DEC_C)]],
                             rb[b], gs[b])

        def pair(j, carry):
            for b in range(2):
                c = 2 * j + b

                def slot(b=b, c=c):
                    pltpu.make_async_copy(
                        a_ref.at[aidx.at[pl.ds(0, DEC_C)]],
                        ra[b], gs[b]).wait()
                    pltpu.make_async_copy(
                        b_ref.at[bidx.at[pl.ds(0, DEC_C)]],
                        rb[b], gs[b]).wait()
                    def addrow(r, c2):
                        for q in range(OUT // 16):
                            plsc.addupdate(rb[b].at[r, pl.ds(q * 16, 16)],
                                           ra[b][r, pl.ds(q * 16, 16)])
                        return c2
                    lax.fori_loop(0, DEC_C, addrow, 0)
                    pltpu.async_copy(
                        rb[b],
                        out_ref.at[pl.ds(base0 + c * DEC_C, DEC_C)],
                        ws[b])
                    nc = c + 2

                    def prefetch():
                        pltpu.async_copy(
                            a_ref.at[aidx.at[pl.ds(nc * DEC_C, DEC_C)]],
                            ra[b], gs[b])
                        pltpu.make_async_copy(
                            rb[b], out_ref.at[pl.ds(base0, DEC_C)],
                            ws[b]).wait()
                        pltpu.async_copy(
                            b_ref.at[bidx.at[pl.ds(nc * DEC_C, DEC_C)]],
                            rb[b], gs[b])
                    pl.when(nc < DEC_CHUNKS)(prefetch)
                pl.when(c < DEC_CHUNKS)(slot)
            return carry
        lax.fori_loop(0, DEC_PAIRS, pair, 0)
        for b in range(2):
            pltpu.make_async_copy(
                rb[b], out_ref.at[pl.ds(base0, DEC_C)], ws[b]).wait()

    f = pl.kernel(
        body,
        out_type=jax.ShapeDtypeStruct((E, OUT), jnp.float32),
        mesh=_sc_mesh(),
        scratch_types=[
            pltpu.VMEM((E // DEC_W,), jnp.int32),
            pltpu.VMEM((E // DEC_W,), jnp.int32),
            pltpu.VMEM((DEC_C, OUT), jnp.float32),
            pltpu.VMEM((DEC_C, OUT), jnp.float32),
            pltpu.VMEM((DEC_C, OUT), jnp.float32),
            pltpu.VMEM((DEC_C, OUT), jnp.float32),
            pltpu.SemaphoreType.DMA,
            pltpu.SemaphoreType.DMA,
            pltpu.SemaphoreType.DMA,
            pltpu.SemaphoreType.DMA,
        ],
    )
    return f(A, Bm, ai3, bi3)


def _collapse(W1aT, W1bT, W2T, W3T, b1r, b2r, b3r):
    def body(w1a, w1b, w2t, w3t, b1_, b2_, b3_, wca, wcb, bc):
        m2 = jnp.dot(w2t[...], w3t[...], preferred_element_type=jnp.float32)
        wca[...] = jnp.dot(w1a[...], m2, preferred_element_type=jnp.float32)
        wcb[...] = jnp.dot(w1b[...], m2, preferred_element_type=jnp.float32)
        bc[...] = (b3_[...]
                   + jnp.dot(b2_[...], w3t[...],
                             preferred_element_type=jnp.float32)
                   + jnp.dot(b1_[...], m2, preferred_element_type=jnp.float32))

    return pl.pallas_call(
        body,
        out_shape=(
            jax.ShapeDtypeStruct((H, OUT), jnp.float32),
            jax.ShapeDtypeStruct((H, OUT), jnp.float32),
            jax.ShapeDtypeStruct((1, OUT), jnp.float32),
        ),
    )(W1aT, W1bT, W2T, W3T, b1r, b2r, b3r)


def _dense_pre1(x, Wro1T, b_rel1r):
    def body(x_, wro, b_, xr):
        xr[...] = (jnp.dot(x_[...], wro[...],
                           preferred_element_type=jnp.float32) + b_[...])

    return pl.pallas_call(
        body,
        grid=(GRID_M,),
        in_specs=[
            pl.BlockSpec((M_TILE, F_IN), lambda i: (i, 0)),
            pl.BlockSpec((F_IN, H), lambda i: (0, 0)),
            pl.BlockSpec((1, H), lambda i: (0, 0)),
        ],
        out_specs=pl.BlockSpec((M_TILE, H), lambda i: (i, 0)),
        out_shape=jax.ShapeDtypeStruct((N, H), jnp.float32),
    )(x, Wro1T, b_rel1r)


def _dense1(g_blocks, xr, Wr1T):
    def body(g0, g1, xr_, wr, h0, h1_, h2_, h3_):
        g = jnp.concatenate([g0[...], g1[...]], axis=1)
        h = jnp.dot(g, wr[...], preferred_element_type=jnp.float32)
        h = jnp.maximum(h + xr_[...], 0.0)
        h0[...] = h[:, 0 * BLK:1 * BLK]
        h1_[...] = h[:, 1 * BLK:2 * BLK]
        h2_[...] = h[:, 2 * BLK:3 * BLK]
        h3_[...] = h[:, 3 * BLK:4 * BLK]

    mb = pl.BlockSpec((M_TILE, BLK), lambda i: (i, 0))
    return pl.pallas_call(
        body,
        grid=(GRID_M,),
        in_specs=[
            mb, mb,
            pl.BlockSpec((M_TILE, H), lambda i: (i, 0)),
            pl.BlockSpec((F_IN, H), lambda i: (0, 0)),
        ],
        out_specs=(mb, mb, mb, mb),
        out_shape=tuple(jax.ShapeDtypeStruct((N, BLK), jnp.float32)
                        for _ in range(4)),
    )(*g_blocks, xr, Wr1T)


def _dense_pre2(h_blocks, Wro2T, WlinT, b_rel2r, b_linr):
    def body(h0, h1_, h2_, h3_, wro2, wlin, br2, bl, p_out, q_out):
        h1 = jnp.concatenate([h0[...], h1_[...], h2_[...], h3_[...]], axis=1)
        p_out[...] = (jnp.dot(h1, wro2[...],
                              preferred_element_type=jnp.float32) + br2[...])
        q_out[...] = (jnp.dot(h1, wlin[...],
                              preferred_element_type=jnp.float32) + bl[...])

    mb = pl.BlockSpec((M_TILE, BLK), lambda i: (i, 0))
    mh = pl.BlockSpec((M_TILE, H), lambda i: (i, 0))
    return pl.pallas_call(
        body,
        grid=(GRID_M,),
        in_specs=[
            mb, mb, mb, mb,
            pl.BlockSpec((H, H), lambda i: (0, 0)),
            pl.BlockSpec((H, H), lambda i: (0, 0)),
            pl.BlockSpec((1, H), lambda i: (0, 0)),
            pl.BlockSpec((1, H), lambda i: (0, 0)),
        ],
        out_specs=(mh, mh),
        out_shape=(jax.ShapeDtypeStruct((N, H), jnp.float32),
                   jax.ShapeDtypeStruct((N, H), jnp.float32)),
    )(*h_blocks, Wro2T, WlinT, b_rel2r, b_linr)


def _dense2(g_blocks, p, q, Wr2T, WlinT_b, WcAT, WcBT, bc):
    def body(g0, g1, g2, g3, p_, q_, wr2, wlb, wca, wcb, bc_, a_out, b_out):
        g = jnp.concatenate([g0[...], g1[...], g2[...], g3[...]], axis=1)
        t = jnp.dot(g, wr2[...], preferred_element_type=jnp.float32)
        h2 = jnp.maximum(t + p_[...], 0.0)
        z = q_[...] + jnp.dot(h2, wlb[...],
                              preferred_element_type=jnp.float32)
        a_out[...] = (jnp.dot(z, wca[...], preferred_element_type=jnp.float32)
                      + bc_[...])
        b_out[...] = jnp.dot(z, wcb[...], preferred_element_type=jnp.float32)

    mb = pl.BlockSpec((M_TILE, BLK), lambda i: (i, 0))
    mh = pl.BlockSpec((M_TILE, H), lambda i: (i, 0))
    mo = pl.BlockSpec((M_TILE, OUT), lambda i: (i, 0))
    wfull = lambda r, c: pl.BlockSpec((r, c), lambda i: (0, 0))
    return pl.pallas_call(
        body,
        grid=(GRID_M,),
        in_specs=[
            mb, mb, mb, mb, mh, mh,
            wfull(H, H), wfull(H, H),
            wfull(H, OUT), wfull(H, OUT),
            wfull(1, OUT),
        ],
        out_specs=(mo, mo),
        out_shape=(jax.ShapeDtypeStruct((N, OUT), jnp.float32),
                   jax.ShapeDtypeStruct((N, OUT), jnp.float32)),
    )(*g_blocks, p, q, Wr2T, WlinT_b, WcAT, WcBT, bc)


def kernel(x, edge_index, edge_label_index, W_rel1, b_rel1, W_root1,
           W_rel2, b_rel2, W_root2, W_lin, b_lin,
           W1, b1, W2, b2, W3, b3):
    src = edge_index[0]
    dst = edge_index[1].reshape(NSUB, SEG_CHUNKS, SEG_C)
    ai = edge_label_index[0]
    bi = edge_label_index[1]
    x0 = x[:, :BLK]
    x1 = x[:, BLK:]
    Wr1T = W_rel1.T
    Wro1T = W_root1.T
    Wr2T = W_rel2.T
    Wro2T = W_root2.T
    WlinT_a = W_lin[:, :H].T
    WlinT_b = W_lin[:, H:].T
    W1aT = W1[:, :H].T
    W1bT = W1[:, H:].T
    W2T = W2.T
    W3T = W3.T

    g1 = _segsum((x0, x1), src, dst)
    WcAT, WcBT, bc = _collapse(W1aT, W1bT, W2T, W3T,
                               b1[None], b2[None], b3[None])
    xr = _dense_pre1(x, Wro1T, b_rel1[None])
    h1b = _dense1(g1, xr, Wr1T)
    g2 = _segsum(h1b, src, dst)
    p, q = _dense_pre2(h1b, Wro2T, WlinT_a, b_rel2[None], b_lin[None])
    A, Bm = _dense2(g2, p, q, Wr2T, WlinT_b, WcAT, WcBT, bc)
    return _decode(A, Bm, ai, bi)


# speedup vs baseline: 1.0189x; 1.0189x over previous
import jax
import jax.numpy as jnp
from jax import lax
from jax.experimental import pallas as pl
from jax.experimental.pallas import tpu as pltpu
from jax.experimental.pallas import tpu_sc as plsc

N = 10000
E = 160000
F_IN = 256
H = 512
OUT = 256
BLK = 128
NSUB = 16

EDGES_PER_SUB = E // NSUB
SEG_C = 80
SEG_CHUNKS = (E // NSUB) // SEG_C
SEG_PAIRS = (SEG_CHUNKS + 1) // 2
ROWS_MAIN = 624
ROWS_TAIL = N - ROWS_MAIN * NSUB

DEC_C = 40
DEC_W = 32
DEC_CHUNKS = (E // DEC_W) // DEC_C
DEC_PAIRS = (DEC_CHUNKS + 1) // 2

M_TILE = 1000
GRID_M = N // M_TILE


def _sc_mesh():
    return plsc.VectorSubcoreMesh(core_axis_name="c", subcore_axis_name="s")


def _segsum(x_blocks, src3, dst3):
    nblk = len(x_blocks)
    per_core = nblk // 2

    def body(*refs):
        x_refs = refs[:nblk]
        src_ref = refs[nblk]
        dst_ref = refs[nblk + 1]
        outs = refs[nblk + 2:2 * nblk + 2]
        acc, sidx, didx, rows0, rows1, sem0, sem1 = refs[2 * nblk + 2:]
        rows = (rows0, rows1)
        sems = (sem0, sem1)
        cid = lax.axis_index("c")
        sid = lax.axis_index("s")
        zvec = jnp.zeros((16,), jnp.float32)

        pltpu.sync_copy(src_ref.at[pl.ds(sid * EDGES_PER_SUB, EDGES_PER_SUB)],
                        sidx)
        pltpu.sync_copy(dst_ref.at[sid], didx)

        def one_block(xb, ob):
            def zrow(r, carry):
                for c8 in range(BLK // 16):
                    rows0[r, pl.ds(c8 * 16, 16)] = zvec
                return carry
            lax.fori_loop(0, SEG_C, zrow, 0)
            r0 = sid * ROWS_MAIN
            off = 0
            while off < ROWS_MAIN:
                span = min(SEG_C, ROWS_MAIN - off)
                pltpu.sync_copy(rows0.at[pl.ds(0, span)],
                                acc.at[pl.ds(r0 + off, span)])
                off += span
            pl.when(sid == 0)(lambda: pltpu.sync_copy(
                rows0.at[pl.ds(0, ROWS_TAIL)],
                acc.at[pl.ds(ROWS_MAIN * NSUB, ROWS_TAIL)]))
            plsc.subcore_barrier()

            for b in range(2):
                pltpu.async_copy(xb.at[sidx.at[pl.ds(b * SEG_C, SEG_C)]],
                                 rows[b], sems[b])

            def pair(j, carry):
                for b in range(2):
                    c = 2 * j + b

                    def slot(b=b, c=c):
                        pltpu.make_async_copy(
                            xb.at[sidx.at[pl.ds(0, SEG_C)]],
                            rows[b], sems[b]).wait()
                        pltpu.sync_copy(rows[b], acc.at[didx.at[c]],
                                        add=True)
                        nc = c + 2

                        def seg_prefetch():
                            pltpu.async_copy(
                                xb.at[sidx.at[pl.ds(nc * SEG_C, SEG_C)]],
                                rows[b], sems[b])
                        pl.when(nc < SEG_CHUNKS)(seg_prefetch)
                    pl.when(c < SEG_CHUNKS)(slot)
                return carry
            lax.fori_loop(0, SEG_PAIRS, pair, 0)
            plsc.subcore_barrier()
            pltpu.sync_copy(acc.at[pl.ds(r0, ROWS_MAIN)],
                            ob.at[pl.ds(r0, ROWS_MAIN)])
            pl.when(sid == 0)(lambda: pltpu.sync_copy(
                acc.at[pl.ds(ROWS_MAIN * NSUB, ROWS_TAIL)],
                ob.at[pl.ds(ROWS_MAIN * NSUB, ROWS_TAIL)]))

        for core in range(2):
            def run_core(core=core):
                for b in range(per_core):
                    blk = core * per_core + b
                    one_block(x_refs[blk], outs[blk])
            pl.when(cid == core)(run_core)

    f = pl.kernel(
        body,
        out_type=tuple(jax.ShapeDtypeStruct((N, BLK), jnp.float32)
                       for _ in range(nblk)),
        mesh=_sc_mesh(),
        scratch_types=[
            pltpu.VMEM_SHARED((N, BLK), jnp.float32),
            pltpu.VMEM((EDGES_PER_SUB,), jnp.int32),
            pltpu.VMEM((SEG_CHUNKS, SEG_C), jnp.int32),
            pltpu.VMEM((SEG_C, BLK), jnp.float32),
            pltpu.VMEM((SEG_C, BLK), jnp.float32),
            pltpu.SemaphoreType.DMA,
            pltpu.SemaphoreType.DMA,
        ],
    )
    return f(*x_blocks, src3, dst3)


def _decode(A, Bm, ai3, bi3):
    span = E // DEC_W

    def body(a_ref, b_ref, ai_ref, bi_ref, out_ref,
             aidx, bidx, ra0, ra1, rb0, rb1, g0, g1, w0, w1):
        cid = lax.axis_index("c")
        sid = lax.axis_index("s")
        wid = sid * 2 + cid
        ra = (ra0, ra1)
        rb = (rb0, rb1)
        gs = (g0, g1)
        ws = (w0, w1)
        base0 = wid * span

        pltpu.sync_copy(ai_ref.at[pl.ds(base0, span)], aidx)
        pltpu.sync_copy(bi_ref.at[pl.ds(base0, span)], bidx)
        for b in range(2):
            pltpu.async_copy(a_ref.at[aidx.at[pl.ds(b * DEC_C, DEC_C)]],
                             ra[b], gs[b])
            pltpu.async_copy(b_ref.at[bidx.at[pl.ds(b * DEC_C, DEC_C)]],
                             rb[b], gs[b])

        def pair(j, carry):
            for b in range(2):
                c = 2 * j + b

                def slot(b=b, c=c):
                    pltpu.make_async_copy(
                        a_ref.at[aidx.at[pl.ds(0, DEC_C)]],
                        ra[b], gs[b]).wait()
                    pltpu.make_async_copy(
                        b_ref.at[bidx.at[pl.ds(0, DEC_C)]],
                        rb[b], gs[b]).wait()
                    def addrow(r, c2):
                        for q in range(OUT // 16):
                            plsc.addupdate(rb[b].at[r, pl.ds(q * 16, 16)],
                                           ra[b][r, pl.ds(q * 16, 16)])
                        return c2
                    lax.fori_loop(0, DEC_C, addrow, 0)
                    pltpu.async_copy(
                        rb[b],
                        out_ref.at[pl.ds(base0 + c * DEC_C, DEC_C)],
                        ws[b])
                    nc = c + 2

                    def prefetch():
                        pltpu.async_copy(
                            a_ref.at[aidx.at[pl.ds(nc * DEC_C, DEC_C)]],
                            ra[b], gs[b])
                        pltpu.make_async_copy(
                            rb[b], out_ref.at[pl.ds(base0, DEC_C)],
                            ws[b]).wait()
                        pltpu.async_copy(
                            b_ref.at[bidx.at[pl.ds(nc * DEC_C, DEC_C)]],
                            rb[b], gs[b])
                    pl.when(nc < DEC_CHUNKS)(prefetch)
                pl.when(c < DEC_CHUNKS)(slot)
            return carry
        lax.fori_loop(0, DEC_PAIRS, pair, 0)
        for b in range(2):
            pltpu.make_async_copy(
                rb[b], out_ref.at[pl.ds(base0, DEC_C)], ws[b]).wait()

    f = pl.kernel(
        body,
        out_type=jax.ShapeDtypeStruct((E, OUT), jnp.float32),
        mesh=_sc_mesh(),
        scratch_types=[
            pltpu.VMEM((E // DEC_W,), jnp.int32),
            pltpu.VMEM((E // DEC_W,), jnp.int32),
            pltpu.VMEM((DEC_C, OUT), jnp.float32),
            pltpu.VMEM((DEC_C, OUT), jnp.float32),
            pltpu.VMEM((DEC_C, OUT), jnp.float32),
            pltpu.VMEM((DEC_C, OUT), jnp.float32),
            pltpu.SemaphoreType.DMA,
            pltpu.SemaphoreType.DMA,
            pltpu.SemaphoreType.DMA,
            pltpu.SemaphoreType.DMA,
        ],
    )
    return f(A, Bm, ai3, bi3)


def _collapse(W1aT, W1bT, W2T, W3T, b1r, b2r, b3r):
    def body(w1a, w1b, w2t, w3t, b1_, b2_, b3_, wca, wcb, bc):
        m2 = jnp.dot(w2t[...], w3t[...], preferred_element_type=jnp.float32)
        wca[...] = jnp.dot(w1a[...], m2, preferred_element_type=jnp.float32)
        wcb[...] = jnp.dot(w1b[...], m2, preferred_element_type=jnp.float32)
        bc[...] = (b3_[...]
                   + jnp.dot(b2_[...], w3t[...],
                             preferred_element_type=jnp.float32)
                   + jnp.dot(b1_[...], m2, preferred_element_type=jnp.float32))

    return pl.pallas_call(
        body,
        out_shape=(
            jax.ShapeDtypeStruct((H, OUT), jnp.float32),
            jax.ShapeDtypeStruct((H, OUT), jnp.float32),
            jax.ShapeDtypeStruct((1, OUT), jnp.float32),
        ),
    )(W1aT, W1bT, W2T, W3T, b1r, b2r, b3r)


def _dense1(g_blocks, x, Wr1T, Wro1T, b_rel1r):
    def body(g0, g1, x_, wr, wro, b_, h0, h1_, h2_, h3_):
        g = jnp.concatenate([g0[...], g1[...]], axis=1)
        h = jnp.dot(g, wr[...], preferred_element_type=jnp.float32)
        h = h + jnp.dot(x_[...], wro[...], preferred_element_type=jnp.float32)
        h = jnp.maximum(h + b_[...], 0.0)
        h0[...] = h[:, 0 * BLK:1 * BLK]
        h1_[...] = h[:, 1 * BLK:2 * BLK]
        h2_[...] = h[:, 2 * BLK:3 * BLK]
        h3_[...] = h[:, 3 * BLK:4 * BLK]

    mb = pl.BlockSpec((M_TILE, BLK), lambda i: (i, 0))
    return pl.pallas_call(
        body,
        grid=(GRID_M,),
        in_specs=[
            mb, mb,
            pl.BlockSpec((M_TILE, F_IN), lambda i: (i, 0)),
            pl.BlockSpec((F_IN, H), lambda i: (0, 0)),
            pl.BlockSpec((F_IN, H), lambda i: (0, 0)),
            pl.BlockSpec((1, H), lambda i: (0, 0)),
        ],
        out_specs=(mb, mb, mb, mb),
        out_shape=tuple(jax.ShapeDtypeStruct((N, BLK), jnp.float32)
                        for _ in range(4)),
    )(*g_blocks, x, Wr1T, Wro1T, b_rel1r)


def _dense2(g_blocks, h_blocks, Wr2T, Wro2T, WlinT_a, WlinT_b,
            WcAT, WcBT, b_rel2r, b_linr, bc):
    def body(g0, g1, g2, g3, h0, h1_, h2_, h3_, wr2, wro2, wla, wlb,
             wca, wcb, br2, bl, bc_, a_out, b_out):
        g = jnp.concatenate([g0[...], g1[...], g2[...], g3[...]], axis=1)
        h1 = jnp.concatenate([h0[...], h1_[...], h2_[...], h3_[...]], axis=1)
        t = jnp.dot(g, wr2[...], preferred_element_type=jnp.float32)
        t = t + jnp.dot(h1, wro2[...], preferred_element_type=jnp.float32)
        h2 = jnp.maximum(t + br2[...], 0.0)
        z = jnp.dot(h1, wla[...], preferred_element_type=jnp.float32)
        z = z + jnp.dot(h2, wlb[...], preferred_element_type=jnp.float32)
        z = z + bl[...]
        a_out[...] = (jnp.dot(z, wca[...], preferred_element_type=jnp.float32)
                      + bc_[...])
        b_out[...] = jnp.dot(z, wcb[...], preferred_element_type=jnp.float32)

    mb = pl.BlockSpec((M_TILE, BLK), lambda i: (i, 0))
    mo = pl.BlockSpec((M_TILE, OUT), lambda i: (i, 0))
    wfull = lambda r, c: pl.BlockSpec((r, c), lambda i: (0, 0))
    return pl.pallas_call(
        body,
        grid=(GRID_M,),
        in_specs=[
            mb, mb, mb, mb, mb, mb, mb, mb,
            wfull(H, H), wfull(H, H), wfull(H, H), wfull(H, H),
            wfull(H, OUT), wfull(H, OUT),
            wfull(1, H), wfull(1, H), wfull(1, OUT),
        ],
        out_specs=(mo, mo),
        out_shape=(jax.ShapeDtypeStruct((N, OUT), jnp.float32),
                   jax.ShapeDtypeStruct((N, OUT), jnp.float32)),
    )(*g_blocks, *h_blocks, Wr2T, Wro2T, WlinT_a, WlinT_b, WcAT, WcBT,
      b_rel2r, b_linr, bc)


def kernel(x, edge_index, edge_label_index, W_rel1, b_rel1, W_root1,
           W_rel2, b_rel2, W_root2, W_lin, b_lin,
           W1, b1, W2, b2, W3, b3):
    src = edge_index[0]
    dst = edge_index[1].reshape(NSUB, SEG_CHUNKS, SEG_C)
    ai = edge_label_index[0]
    bi = edge_label_index[1]
    x0 = x[:, :BLK]
    x1 = x[:, BLK:]
    Wr1T = W_rel1.T
    Wro1T = W_root1.T
    Wr2T = W_rel2.T
    Wro2T = W_root2.T
    WlinT_a = W_lin[:, :H].T
    WlinT_b = W_lin[:, H:].T
    W1aT = W1[:, :H].T
    W1bT = W1[:, H:].T
    W2T = W2.T
    W3T = W3.T

    WcAT, WcBT, bc = _collapse(W1aT, W1bT, W2T, W3T,
                               b1[None], b2[None], b3[None])
    g1 = _segsum((x0, x1), src, dst)
    h1b = _dense1(g1, x, Wr1T, Wro1T, b_rel1[None])
    g2 = _segsum(h1b, src, dst)
    A, Bm = _dense2(g2, h1b, Wr2T, Wro2T, WlinT_a, WlinT_b, WcAT, WcBT,
                    b_rel2[None], b_lin[None], bc)
    return _decode(A, Bm, ai, bi)


# speedup vs baseline: 1.0226x; 1.0037x over previous
import jax
import jax.numpy as jnp
from jax import lax
from jax.experimental import pallas as pl
from jax.experimental.pallas import tpu as pltpu
from jax.experimental.pallas import tpu_sc as plsc

N = 10000
E = 160000
F_IN = 256
H = 512
OUT = 256
BLK = 128
NSUB = 16

EDGES_PER_SUB = E // NSUB
SEG_C = 80
SEG_CHUNKS = (E // NSUB) // SEG_C
SEG_PAIRS = (SEG_CHUNKS + 1) // 2
ROWS_MAIN = 624
ROWS_TAIL = N - ROWS_MAIN * NSUB

DEC_C = 40
DEC_W = 32
DEC_CHUNKS = (E // DEC_W) // DEC_C
DEC_PAIRS = (DEC_CHUNKS + 1) // 2

M_TILE = 2000
GRID_M = N // M_TILE


def _sc_mesh():
    return plsc.VectorSubcoreMesh(core_axis_name="c", subcore_axis_name="s")


def _segsum(x_blocks, src3, dst3):
    nblk = len(x_blocks)
    per_core = nblk // 2

    def body(*refs):
        x_refs = refs[:nblk]
        src_ref = refs[nblk]
        dst_ref = refs[nblk + 1]
        outs = refs[nblk + 2:2 * nblk + 2]
        acc, sidx, didx, rows0, rows1, sem0, sem1 = refs[2 * nblk + 2:]
        rows = (rows0, rows1)
        sems = (sem0, sem1)
        cid = lax.axis_index("c")
        sid = lax.axis_index("s")
        zvec = jnp.zeros((16,), jnp.float32)

        pltpu.sync_copy(src_ref.at[pl.ds(sid * EDGES_PER_SUB, EDGES_PER_SUB)],
                        sidx)
        pltpu.sync_copy(dst_ref.at[sid], didx)

        def one_block(xb, ob):
            def zrow(r, carry):
                for c8 in range(BLK // 16):
                    rows0[r, pl.ds(c8 * 16, 16)] = zvec
                return carry
            lax.fori_loop(0, SEG_C, zrow, 0)
            r0 = sid * ROWS_MAIN
            off = 0
            while off < ROWS_MAIN:
                span = min(SEG_C, ROWS_MAIN - off)
                pltpu.sync_copy(rows0.at[pl.ds(0, span)],
                                acc.at[pl.ds(r0 + off, span)])
                off += span
            pl.when(sid == 0)(lambda: pltpu.sync_copy(
                rows0.at[pl.ds(0, ROWS_TAIL)],
                acc.at[pl.ds(ROWS_MAIN * NSUB, ROWS_TAIL)]))
            plsc.subcore_barrier()

            for b in range(2):
                pltpu.async_copy(xb.at[sidx.at[pl.ds(b * SEG_C, SEG_C)]],
                                 rows[b], sems[b])

            def pair(j, carry):
                for b in range(2):
                    c = 2 * j + b

                    def slot(b=b, c=c):
                        pltpu.make_async_copy(
                            xb.at[sidx.at[pl.ds(0, SEG_C)]],
                            rows[b], sems[b]).wait()
                        pltpu.sync_copy(rows[b], acc.at[didx.at[c]],
                                        add=True)
                        nc = c + 2

                        def seg_prefetch():
                            pltpu.async_copy(
                                xb.at[sidx.at[pl.ds(nc * SEG_C, SEG_C)]],
                                rows[b], sems[b])
                        pl.when(nc < SEG_CHUNKS)(seg_prefetch)
                    pl.when(c < SEG_CHUNKS)(slot)
                return carry
            lax.fori_loop(0, SEG_PAIRS, pair, 0)
            plsc.subcore_barrier()
            pltpu.sync_copy(acc.at[pl.ds(r0, ROWS_MAIN)],
                            ob.at[pl.ds(r0, ROWS_MAIN)])
            pl.when(sid == 0)(lambda: pltpu.sync_copy(
                acc.at[pl.ds(ROWS_MAIN * NSUB, ROWS_TAIL)],
                ob.at[pl.ds(ROWS_MAIN * NSUB, ROWS_TAIL)]))

        for core in range(2):
            def run_core(core=core):
                for b in range(per_core):
                    blk = core * per_core + b
                    one_block(x_refs[blk], outs[blk])
            pl.when(cid == core)(run_core)

    f = pl.kernel(
        body,
        out_type=tuple(jax.ShapeDtypeStruct((N, BLK), jnp.float32)
                       for _ in range(nblk)),
        mesh=_sc_mesh(),
        scratch_types=[
            pltpu.VMEM_SHARED((N, BLK), jnp.float32),
            pltpu.VMEM((EDGES_PER_SUB,), jnp.int32),
            pltpu.VMEM((SEG_CHUNKS, SEG_C), jnp.int32),
            pltpu.VMEM((SEG_C, BLK), jnp.float32),
            pltpu.VMEM((SEG_C, BLK), jnp.float32),
            pltpu.SemaphoreType.DMA,
            pltpu.SemaphoreType.DMA,
        ],
    )
    return f(*x_blocks, src3, dst3)


def _decode(A, Bm, ai3, bi3):
    span = E // DEC_W

    def body(a_ref, b_ref, ai_ref, bi_ref, out_ref,
             aidx, bidx, ra0, ra1, rb0, rb1, g0, g1, w0, w1):
        cid = lax.axis_index("c")
        sid = lax.axis_index("s")
        wid = sid * 2 + cid
        ra = (ra0, ra1)
        rb = (rb0, rb1)
        gs = (g0, g1)
        ws = (w0, w1)
        base0 = wid * span

        pltpu.sync_copy(ai_ref.at[pl.ds(base0, span)], aidx)
        pltpu.sync_copy(bi_ref.at[pl.ds(base0, span)], bidx)
        for b in range(2):
            pltpu.async_copy(a_ref.at[aidx.at[pl.ds(b * DEC_C, DEC_C)]],
                             ra[b], gs[b])
            pltpu.async_copy(b_ref.at[bidx.at[pl.ds(b * DEC_C, DEC_C)]],
                             rb[b], gs[b])

        def pair(j, carry):
            for b in range(2):
                c = 2 * j + b

                def slot(b=b, c=c):
                    pltpu.make_async_copy(
                        a_ref.at[aidx.at[pl.ds(0, DEC_C)]],
                        ra[b], gs[b]).wait()
                    pltpu.make_async_copy(
                        b_ref.at[bidx.at[pl.ds(0, DEC_C)]],
                        rb[b], gs[b]).wait()
                    def addrow(r, c2):
                        for q in range(OUT // 16):
                            plsc.addupdate(rb[b].at[r, pl.ds(q * 16, 16)],
                                           ra[b][r, pl.ds(q * 16, 16)])
                        return c2
                    lax.fori_loop(0, DEC_C, addrow, 0)
                    pltpu.async_copy(
                        rb[b],
                        out_ref.at[pl.ds(base0 + c * DEC_C, DEC_C)],
                        ws[b])
                    nc = c + 2

                    def prefetch():
                        pltpu.async_copy(
                            a_ref.at[aidx.at[pl.ds(nc * DEC_C, DEC_C)]],
                            ra[b], gs[b])
                        pltpu.make_async_copy(
                            rb[b], out_ref.at[pl.ds(base0, DEC_C)],
                            ws[b]).wait()
                        pltpu.async_copy(
                            b_ref.at[bidx.at[pl.ds(nc * DEC_C, DEC_C)]],
                            rb[b], gs[b])
                    pl.when(nc < DEC_CHUNKS)(prefetch)
                pl.when(c < DEC_CHUNKS)(slot)
            return carry
        lax.fori_loop(0, DEC_PAIRS, pair, 0)
        for b in range(2):
            pltpu.make_async_copy(
                rb[b], out_ref.at[pl.ds(base0, DEC_C)], ws[b]).wait()

    f = pl.kernel(
        body,
        out_type=jax.ShapeDtypeStruct((E, OUT), jnp.float32),
        mesh=_sc_mesh(),
        scratch_types=[
            pltpu.VMEM((E // DEC_W,), jnp.int32),
            pltpu.VMEM((E // DEC_W,), jnp.int32),
            pltpu.VMEM((DEC_C, OUT), jnp.float32),
            pltpu.VMEM((DEC_C, OUT), jnp.float32),
            pltpu.VMEM((DEC_C, OUT), jnp.float32),
            pltpu.VMEM((DEC_C, OUT), jnp.float32),
            pltpu.SemaphoreType.DMA,
            pltpu.SemaphoreType.DMA,
            pltpu.SemaphoreType.DMA,
            pltpu.SemaphoreType.DMA,
        ],
    )
    return f(A, Bm, ai3, bi3)


def _dense1(g_blocks, x, Wr1T, Wro1T, b_rel1r):
    def body(g0, g1, x_, wr, wro, b_, h0, h1_, h2_, h3_):
        g = jnp.concatenate([g0[...], g1[...]], axis=1)
        h = jnp.dot(g, wr[...], preferred_element_type=jnp.float32)
        h = h + jnp.dot(x_[...], wro[...], preferred_element_type=jnp.float32)
        h = jnp.maximum(h + b_[...], 0.0)
        h0[...] = h[:, 0 * BLK:1 * BLK]
        h1_[...] = h[:, 1 * BLK:2 * BLK]
        h2_[...] = h[:, 2 * BLK:3 * BLK]
        h3_[...] = h[:, 3 * BLK:4 * BLK]

    mb = pl.BlockSpec((M_TILE, BLK), lambda i: (i, 0))
    return pl.pallas_call(
        body,
        grid=(GRID_M,),
        in_specs=[
            mb, mb,
            pl.BlockSpec((M_TILE, F_IN), lambda i: (i, 0)),
            pl.BlockSpec((F_IN, H), lambda i: (0, 0)),
            pl.BlockSpec((F_IN, H), lambda i: (0, 0)),
            pl.BlockSpec((1, H), lambda i: (0, 0)),
        ],
        out_specs=(mb, mb, mb, mb),
        out_shape=tuple(jax.ShapeDtypeStruct((N, BLK), jnp.float32)
                        for _ in range(4)),
    )(*g_blocks, x, Wr1T, Wro1T, b_rel1r)


def _dense2(g_blocks, h_blocks, Wr2T, Wro2T, WlinT_a, WlinT_b,
            W1aT, W1bT, W2T, W3T, b_rel2r, b_linr, b1r, b2r, b3r):
    def body(g0, g1, g2, g3, h0, h1_, h2_, h3_, wr2, wro2, wla, wlb,
             w1a, w1b, w2t, w3t, br2, bl, b1_, b2_, b3_, a_out, b_out,
             wca, wcb, bc_):
        @pl.when(pl.program_id(0) == 0)
        def _():
            m2 = jnp.dot(w2t[...], w3t[...],
                         preferred_element_type=jnp.float32)
            wca[...] = jnp.dot(w1a[...], m2,
                               preferred_element_type=jnp.float32)
            wcb[...] = jnp.dot(w1b[...], m2,
                               preferred_element_type=jnp.float32)
            bc_[...] = (b3_[...]
                        + jnp.dot(b2_[...], w3t[...],
                                  preferred_element_type=jnp.float32)
                        + jnp.dot(b1_[...], m2,
                                  preferred_element_type=jnp.float32))

        g = jnp.concatenate([g0[...], g1[...], g2[...], g3[...]], axis=1)
        h1 = jnp.concatenate([h0[...], h1_[...], h2_[...], h3_[...]], axis=1)
        t = jnp.dot(g, wr2[...], preferred_element_type=jnp.float32)
        t = t + jnp.dot(h1, wro2[...], preferred_element_type=jnp.float32)
        h2 = jnp.maximum(t + br2[...], 0.0)
        z = jnp.dot(h1, wla[...], preferred_element_type=jnp.float32)
        z = z + jnp.dot(h2, wlb[...], preferred_element_type=jnp.float32)
        z = z + bl[...]
        a_out[...] = (jnp.dot(z, wca[...], preferred_element_type=jnp.float32)
                      + bc_[...])
        b_out[...] = jnp.dot(z, wcb[...], preferred_element_type=jnp.float32)

    mb = pl.BlockSpec((M_TILE, BLK), lambda i: (i, 0))
    mo = pl.BlockSpec((M_TILE, OUT), lambda i: (i, 0))
    wfull = lambda r, c: pl.BlockSpec((r, c), lambda i: (0, 0))
    return pl.pallas_call(
        body,
        grid=(GRID_M,),
        in_specs=[
            mb, mb, mb, mb, mb, mb, mb, mb,
            wfull(H, H), wfull(H, H), wfull(H, H), wfull(H, H),
            wfull(H, H), wfull(H, H), wfull(H, H), wfull(H, OUT),
            wfull(1, H), wfull(1, H),
            wfull(1, H), wfull(1, H), wfull(1, OUT),
        ],
        out_specs=(mo, mo),
        out_shape=(jax.ShapeDtypeStruct((N, OUT), jnp.float32),
                   jax.ShapeDtypeStruct((N, OUT), jnp.float32)),
        scratch_shapes=[
            pltpu.VMEM((H, OUT), jnp.float32),
            pltpu.VMEM((H, OUT), jnp.float32),
            pltpu.VMEM((1, OUT), jnp.float32),
        ],
    )(*g_blocks, *h_blocks, Wr2T, Wro2T, WlinT_a, WlinT_b,
      W1aT, W1bT, W2T, W3T, b_rel2r, b_linr, b1r, b2r, b3r)


def kernel(x, edge_index, edge_label_index, W_rel1, b_rel1, W_root1,
           W_rel2, b_rel2, W_root2, W_lin, b_lin,
           W1, b1, W2, b2, W3, b3):
    src = edge_index[0]
    dst = edge_index[1].reshape(NSUB, SEG_CHUNKS, SEG_C)
    ai = edge_label_index[0]
    bi = edge_label_index[1]
    x0 = x[:, :BLK]
    x1 = x[:, BLK:]
    Wr1T = W_rel1.T
    Wro1T = W_root1.T
    Wr2T = W_rel2.T
    Wro2T = W_root2.T
    WlinT_a = W_lin[:, :H].T
    WlinT_b = W_lin[:, H:].T
    W1aT = W1[:, :H].T
    W1bT = W1[:, H:].T
    W2T = W2.T
    W3T = W3.T

    g1 = _segsum((x0, x1), src, dst)
    h1b = _dense1(g1, x, Wr1T, Wro1T, b_rel1[None])
    g2 = _segsum(h1b, src, dst)
    A, Bm = _dense2(g2, h1b, Wr2T, Wro2T, WlinT_a, WlinT_b,
                    W1aT, W1bT, W2T, W3T,
                    b_rel2[None], b_lin[None], b1[None], b2[None], b3[None])
    return _decode(A, Bm, ai, bi)
